# baseline (speedup 1.0000x reference)
import functools

import jax
import jax.numpy as jnp
from jax import lax
from jax.experimental import pallas as pl
from jax.experimental.pallas import tpu as pltpu
from jax.experimental.pallas import tpu_sc as plsc

N = 10000
E = 320000
F_IN = 128
HID = 16
C = 10
G = 64

NC = 2
NS = 16
NW = NC * NS
EW = E // NW
K = 2000
CH = EW // K
SB = 624
REM = N - NS * SB
ROFF = NS * SB

PK = 128 // HID
NP = N // PK
_RB = 2000
_RP = _RB // PK
_GRID = N // _RB

_sc_mesh = plsc.VectorSubcoreMesh(core_axis_name="c", subcore_axis_name="s")
_sc_params = pltpu.CompilerParams(use_tc_tiling_on_sc=False,
                                  needs_layout_passes=False)


def _zero_acc_slice(zbuf, acc, s):
    def body(i, carry):
        zbuf[i, :] = jnp.zeros((HID,), jnp.float32)
        return carry
    lax.fori_loop(0, SB, body, 0)
    pltpu.sync_copy(zbuf.at[pl.ds(0, SB)], acc.at[pl.ds(s * SB, SB)])

    @pl.when(s == NS - 1)
    def _rem():
        pltpu.sync_copy(zbuf.at[pl.ds(0, REM)], acc.at[pl.ds(ROFF, REM)])


RPS = SB // PK
RREM = REM // PK
RTOT = N // PK // NS


def _repack_out(abuf, stage, out_hbm, c, s, nmine):
    def body(i, carry):
        row = abuf[i, :]
        stage[i // PK, pl.ds((i % PK) * HID, HID)] = row
        return carry
    lax.fori_loop(0, nmine, body, 0)

    pltpu.sync_copy(stage.at[pl.ds(0, RPS)],
                    out_hbm.at[c, pl.ds(s * RPS, RPS)])

    @pl.when(s == NS - 1)
    def _rem():
        pltpu.sync_copy(stage.at[pl.ds(RPS, RREM)],
                        out_hbm.at[c, pl.ds(NS * RPS, RREM)])


@functools.partial(
    pl.kernel,
    out_type=jax.ShapeDtypeStruct((NC, NP, 128), jnp.float32),
    mesh=_sc_mesh,
    compiler_params=_sc_params,
    scratch_types=[
        pltpu.VMEM((K,), jnp.int32),
        pltpu.VMEM((K,), jnp.int32),
        pltpu.VMEM((K,), jnp.int32),
        pltpu.VMEM((K,), jnp.int32),
        pltpu.VMEM((K, HID), jnp.float32),
        pltpu.VMEM((K, HID), jnp.float32),
        pltpu.VMEM((SB + REM, HID), jnp.float32),
        pltpu.VMEM((RPS + RREM, 128), jnp.float32),
        pltpu.VMEM_SHARED((N, HID), jnp.float32),
        pltpu.SemaphoreType.DMA,
        pltpu.SemaphoreType.DMA,
        pltpu.SemaphoreType.DMA,
        pltpu.SemaphoreType.DMA,
        pltpu.SemaphoreType.DMA,
        pltpu.SemaphoreType.DMA,
    ],
)
def _sc_agg(ei_hbm, g_hbm, out_hbm, si0, si1, di0, di1, r0, r1, zbuf, stage,
            acc, ss0, ss1, sd0, sd1, sg0, sg1):
    c = lax.axis_index("c")
    s = lax.axis_index("s")
    wid = s * NC + c
    base = wid * EW
    SI, DI, R = [si0, si1], [di0, di1], [r0, r1]
    SS, SD, SG = [ss0, ss1], [sd0, sd1], [sg0, sg1]

    def start_idx(k):
        b = k & 1
        return (pltpu.async_copy(ei_hbm.at[0, pl.ds(base + k * K, K)],
                                 SI[b], SS[b]),
                pltpu.async_copy(ei_hbm.at[1, pl.ds(base + k * K, K)],
                                 DI[b], SD[b]))

    idx_d = {0: start_idx(0), 1: start_idx(1)}
    idx_d[0][0].wait()
    g_d = {0: pltpu.async_copy(g_hbm.at[SI[0]], R[0], SG[0])}
    _zero_acc_slice(zbuf, acc, s)
    plsc.subcore_barrier()

    for k in range(CH):
        b = k & 1
        if k + 1 < CH:
            idx_d[k + 1][0].wait()
            g_d[k + 1] = pltpu.async_copy(g_hbm.at[SI[1 - b]], R[1 - b],
                                          SG[1 - b])
        g_d[k].wait()
        idx_d[k][1].wait()
        pltpu.sync_copy(R[b], acc.at[DI[b]], add=True)
        if k + 2 < CH:
            idx_d[k + 2] = start_idx(k + 2)

    plsc.subcore_barrier()

    nmine = jnp.where(s == NS - 1, SB + REM, SB)
    pltpu.sync_copy(acc.at[pl.ds(s * SB, SB)], zbuf.at[pl.ds(0, SB)])

    @pl.when(s == NS - 1)
    def _remc():
        pltpu.sync_copy(acc.at[pl.ds(ROFF, REM)], zbuf.at[pl.ds(SB, REM)])

    _repack_out(zbuf, stage, out_hbm, c, s, nmine)


@functools.partial(
    pl.kernel,
    out_type=jax.ShapeDtypeStruct((NC, NP, 128), jnp.float32),
    mesh=_sc_mesh,
    compiler_params=_sc_params,
    scratch_types=[
        pltpu.VMEM((K,), jnp.int32),
        pltpu.VMEM((K,), jnp.int32),
        pltpu.VMEM((K,), jnp.float32),
        pltpu.VMEM((SB + REM,), jnp.float32),
        pltpu.VMEM((RPS + RREM, 128), jnp.float32),
        pltpu.VMEM_SHARED((N,), jnp.float32),
        pltpu.SemaphoreType.DMA,
        pltpu.SemaphoreType.DMA,
    ],
)
def _sc_deg(ei_hbm, out_hbm, di0, di1, ones, dbuf, stage, acc, sd0, sd1):
    c = lax.axis_index("c")
    s = lax.axis_index("s")
    wid = s * NC + c
    base = wid * EW
    DI, SD = [di0, di1], [sd0, sd1]

    def start_idx(k):
        b = k & 1
        return pltpu.async_copy(ei_hbm.at[1, pl.ds(base + k * K, K)],
                                DI[b], SD[b])

    idx_d = {0: start_idx(0), 1: start_idx(1)}

    def fill(i, carry):
        ones[pl.ds(i * 16, 16)] = jnp.full((16,), 1.0, jnp.float32)
        return carry
    lax.fori_loop(0, K // 16, fill, 0)

    def zero(i, carry):
        dbuf[pl.ds(i * 16, 16)] = jnp.zeros((16,), jnp.float32)
        return carry
    lax.fori_loop(0, (SB + REM) // 16, zero, 0)
    pltpu.sync_copy(dbuf.at[pl.ds(0, SB)], acc.at[pl.ds(s * SB, SB)])

    @pl.when(s == NS - 1)
    def _remz():
        pltpu.sync_copy(dbuf.at[pl.ds(0, REM)], acc.at[pl.ds(ROFF, REM)])

    plsc.subcore_barrier()

    for k in range(CH):
        b = k & 1
        idx_d[k].wait()
        pltpu.sync_copy(ones, acc.at[DI[b]], add=True)
        if k + 2 < CH:
            idx_d[k + 2] = start_idx(k + 2)

    plsc.subcore_barrier()

    nmine = jnp.where(s == NS - 1, SB + REM, SB)
    pltpu.sync_copy(acc.at[pl.ds(s * SB, SB)], dbuf.at[pl.ds(0, SB)])

    @pl.when(s == NS - 1)
    def _remc():
        pltpu.sync_copy(acc.at[pl.ds(ROFF, REM)], dbuf.at[pl.ds(SB, REM)])

    def repl(i, carry):
        row = plsc.load_gather(dbuf, [jnp.full((16,), i, jnp.int32)])
        stage[i // PK, pl.ds((i % PK) * HID, HID)] = row
        return carry
    lax.fori_loop(0, nmine, repl, 0)

    pltpu.sync_copy(stage.at[pl.ds(0, RPS)],
                    out_hbm.at[c, pl.ds(s * RPS, RPS)])

    @pl.when(s == NS - 1)
    def _remo():
        pltpu.sync_copy(stage.at[pl.ds(RPS, RREM)],
                        out_hbm.at[c, pl.ds(NS * RPS, RREM)])


def _bdiag_small(w):
    w128 = jnp.concatenate([w] * PK, axis=1)
    colb = lax.broadcasted_iota(jnp.int32, (HID, 128), 1) // HID
    rows = [jnp.where(colb == p, w128, 0.0) for p in range(PK)]
    return jnp.concatenate(rows, axis=0)


def _bdiag_first(w):
    w128 = jnp.concatenate([w] * PK, axis=1)
    colb = lax.broadcasted_iota(jnp.int32, (F_IN, 128), 1) // HID
    rows = [jnp.where(colb == p, w128, 0.0) for p in range(PK)]
    return jnp.concatenate(rows, axis=0)


def _tc_mm(x3, W1):
    def body(x_ref, w_ref, u_ref):
        wbig = _bdiag_first(w_ref[...])
        u_ref[...] = jnp.dot(x_ref[...], wbig,
                             preferred_element_type=jnp.float32)
    return pl.pallas_call(
        body,
        out_shape=jax.ShapeDtypeStruct((NP, 128), jnp.float32),
    )(x3, W1)


def _tc_dinv(u, degp):
    def body(u_ref, dp_ref, dinv_ref, g_ref):
        deg = dp_ref[0] + dp_ref[1] + 1.0
        dinv = lax.rsqrt(deg)
        dinv_ref[...] = dinv
        g_ref[...] = dinv * u_ref[...]
    return pl.pallas_call(
        body,
        out_shape=[
            jax.ShapeDtypeStruct((NP, 128), jnp.float32),
            jax.ShapeDtypeStruct((NP, 128), jnp.float32),
        ],
    )(u, degp)


def _tc_layer(aggp, g, dinv, b, W):
    def body(ap_hbm, g_ref, dinv_ref, b_ref, w_ref, out_ref, ap_vmem, sem):
        cp = pltpu.make_async_copy(ap_hbm, ap_vmem, sem)
        cp.start()
        cp.wait()
        dinv = dinv_ref[...]
        b128 = jnp.concatenate([b_ref[...]] * PK, axis=1)
        h = dinv * (ap_vmem[0] + ap_vmem[1] + g_ref[...]) + b128
        h = jnp.maximum(h, 0.0)
        wbd = _bdiag_small(w_ref[...])
        out_ref[...] = dinv * jnp.dot(h, wbd,
                                      preferred_element_type=jnp.float32)
    return pl.pallas_call(
        body,
        in_specs=[
            pl.BlockSpec(memory_space=pl.ANY),
            pl.BlockSpec((NP, 128), lambda: (0, 0)),
            pl.BlockSpec((NP, 128), lambda: (0, 0)),
            pl.BlockSpec((1, HID), lambda: (0, 0)),
            pl.BlockSpec((HID, HID), lambda: (0, 0)),
        ],
        out_specs=pl.BlockSpec((NP, 128), lambda: (0, 0)),
        out_shape=jax.ShapeDtypeStruct((NP, 128), jnp.float32),
        scratch_shapes=[pltpu.VMEM((NC, NP, 128), jnp.float32),
                        pltpu.SemaphoreType.DMA],
    )(aggp, g, dinv, b, W)


def _tc_final(aggp, g, dinv, b, batch2, Wl, bl):
    def body(ap_ref, g_ref, dinv_ref, b_ref, bt_ref, wl_ref, bl_ref, out_ref):
        b128 = jnp.concatenate([b_ref[...]] * PK, axis=1)
        h3 = dinv_ref[...] * (ap_ref[0] + ap_ref[1] + g_ref[...]) + b128
        pooled = jnp.zeros((G, HID), jnp.float32)
        for p in range(PK):
            hp = h3[:, p * HID:(p + 1) * HID]
            bp = bt_ref[:, p:p + 1]
            seg = (bp == lax.broadcasted_iota(jnp.int32, (NP, G), 1)
                   ).astype(jnp.float32)
            pooled += lax.dot_general(seg, hp, (((0,), (0,)), ((), ())),
                                      preferred_element_type=jnp.float32)
        out_ref[...] = jnp.dot(pooled, wl_ref[...],
                               preferred_element_type=jnp.float32) + bl_ref[...]

    return pl.pallas_call(
        body,
        out_shape=jax.ShapeDtypeStruct((G, C), jnp.float32),
    )(aggp, g, dinv, b, batch2, Wl, bl)


def kernel(x, edge_index, batch, W1, b1, W2, b2, W3, b3, Wl, bl):
    degp_p = _sc_deg(edge_index)
    x3 = x.reshape(NP, PK * F_IN)
    u1_p = _tc_mm(x3, W1)
    dinv_p, g1_p = _tc_dinv(u1_p, degp_p)

    a1 = _sc_agg(edge_index, g1_p.reshape(N, HID))
    g2_p = _tc_layer(a1, g1_p, dinv_p, b1.reshape(1, HID), W2)

    a2 = _sc_agg(edge_index, g2_p.reshape(N, HID))
    g3_p = _tc_layer(a2, g2_p, dinv_p, b2.reshape(1, HID), W3)

    a3 = _sc_agg(edge_index, g3_p.reshape(N, HID))
    out = _tc_final(a3, g3_p, dinv_p,
                    b3.reshape(1, HID), batch.reshape(NP, PK),
                    Wl, bl.reshape(1, C))
    return out


# speedup vs baseline: 1.0157x; 1.0157x over previous
import functools

import jax
import jax.numpy as jnp
from jax import lax
from jax.experimental import pallas as pl
from jax.experimental.pallas import tpu as pltpu
from jax.experimental.pallas import tpu_sc as plsc

N = 10000
E = 320000
F_IN = 128
HID = 16
C = 10
G = 64

NC = 2
NS = 16
NW = NC * NS
EW = E // NW
K = 2000
CH = EW // K
SB = 624
REM = N - NS * SB
ROFF = NS * SB

PK = 128 // HID
NP = N // PK
_RB = 2000
_RP = _RB // PK
_GRID = N // _RB

_sc_mesh = plsc.VectorSubcoreMesh(core_axis_name="c", subcore_axis_name="s")
_sc_params = pltpu.CompilerParams(use_tc_tiling_on_sc=False,
                                  needs_layout_passes=False)


def _zero_acc_slice(zbuf, acc, s):
    def body(i, carry):
        zbuf[i, :] = jnp.zeros((HID,), jnp.float32)
        return carry
    lax.fori_loop(0, SB, body, 0)
    pltpu.sync_copy(zbuf.at[pl.ds(0, SB)], acc.at[pl.ds(s * SB, SB)])

    @pl.when(s == NS - 1)
    def _rem():
        pltpu.sync_copy(zbuf.at[pl.ds(0, REM)], acc.at[pl.ds(ROFF, REM)])


RPS = SB // PK
RREM = REM // PK
RTOT = N // PK // NS


def _repack_out(abuf, stage, out_hbm, c, s, nmine):
    def body(i, carry):
        row = abuf[i, :]
        stage[pl.ds((i // PK) * 128 + (i % PK) * HID, HID)] = row
        return carry
    lax.fori_loop(0, nmine, body, 0)

    pltpu.sync_copy(stage.at[pl.ds(0, RPS * 128)],
                    out_hbm.at[pl.ds(c * (NP * 128) + s * (RPS * 128),
                                     RPS * 128)])

    @pl.when(s == NS - 1)
    def _rem():
        pltpu.sync_copy(stage.at[pl.ds(RPS * 128, RREM * 128)],
                        out_hbm.at[pl.ds(c * (NP * 128) + NS * (RPS * 128),
                                         RREM * 128)])


@functools.partial(
    pl.kernel,
    out_type=jax.ShapeDtypeStruct((NC * NP * 128,), jnp.float32),
    mesh=_sc_mesh,
    compiler_params=_sc_params,
    scratch_types=[
        pltpu.VMEM((K,), jnp.int32),
        pltpu.VMEM((K,), jnp.int32),
        pltpu.VMEM((K,), jnp.int32),
        pltpu.VMEM((K,), jnp.int32),
        pltpu.VMEM((K, HID), jnp.float32),
        pltpu.VMEM((K, HID), jnp.float32),
        pltpu.VMEM((SB + REM, HID), jnp.float32),
        pltpu.VMEM(((RPS + RREM) * 128,), jnp.float32),
        pltpu.VMEM_SHARED((N, HID), jnp.float32),
        pltpu.SemaphoreType.DMA,
        pltpu.SemaphoreType.DMA,
        pltpu.SemaphoreType.DMA,
        pltpu.SemaphoreType.DMA,
        pltpu.SemaphoreType.DMA,
        pltpu.SemaphoreType.DMA,
    ],
)
def _sc_agg(ei_hbm, g_hbm, out_hbm, si0, si1, di0, di1, r0, r1, zbuf, stage,
            acc, ss0, ss1, sd0, sd1, sg0, sg1):
    c = lax.axis_index("c")
    s = lax.axis_index("s")
    wid = s * NC + c
    base = wid * EW
    SI, DI, R = [si0, si1], [di0, di1], [r0, r1]
    SS, SD, SG = [ss0, ss1], [sd0, sd1], [sg0, sg1]

    def start_idx(k):
        b = k & 1
        return (pltpu.async_copy(ei_hbm.at[0, pl.ds(base + k * K, K)],
                                 SI[b], SS[b]),
                pltpu.async_copy(ei_hbm.at[1, pl.ds(base + k * K, K)],
                                 DI[b], SD[b]))

    idx_d = {0: start_idx(0), 1: start_idx(1)}
    idx_d[0][0].wait()
    g_d = {0: pltpu.async_copy(g_hbm.at[SI[0]], R[0], SG[0])}
    _zero_acc_slice(zbuf, acc, s)
    plsc.subcore_barrier()

    for k in range(CH):
        b = k & 1
        if k + 1 < CH:
            idx_d[k + 1][0].wait()
            g_d[k + 1] = pltpu.async_copy(g_hbm.at[SI[1 - b]], R[1 - b],
                                          SG[1 - b])
        g_d[k].wait()
        idx_d[k][1].wait()
        pltpu.sync_copy(R[b], acc.at[DI[b]], add=True)
        if k + 2 < CH:
            idx_d[k + 2] = start_idx(k + 2)

    plsc.subcore_barrier()

    nmine = jnp.where(s == NS - 1, SB + REM, SB)
    pltpu.sync_copy(acc.at[pl.ds(s * SB, SB)], zbuf.at[pl.ds(0, SB)])

    @pl.when(s == NS - 1)
    def _remc():
        pltpu.sync_copy(acc.at[pl.ds(ROFF, REM)], zbuf.at[pl.ds(SB, REM)])

    _repack_out(zbuf, stage, out_hbm, c, s, nmine)


@functools.partial(
    pl.kernel,
    out_type=jax.ShapeDtypeStruct((NC * NP * 128,), jnp.float32),
    mesh=_sc_mesh,
    compiler_params=_sc_params,
    scratch_types=[
        pltpu.VMEM((K,), jnp.int32),
        pltpu.VMEM((K,), jnp.int32),
        pltpu.VMEM((K,), jnp.float32),
        pltpu.VMEM((SB + REM,), jnp.float32),
        pltpu.VMEM(((RPS + RREM) * 128,), jnp.float32),
        pltpu.VMEM_SHARED((N,), jnp.float32),
        pltpu.SemaphoreType.DMA,
        pltpu.SemaphoreType.DMA,
    ],
)
def _sc_deg(ei_hbm, out_hbm, di0, di1, ones, dbuf, stage, acc, sd0, sd1):
    c = lax.axis_index("c")
    s = lax.axis_index("s")
    wid = s * NC + c
    base = wid * EW
    DI, SD = [di0, di1], [sd0, sd1]

    def start_idx(k):
        b = k & 1
        return pltpu.async_copy(ei_hbm.at[1, pl.ds(base + k * K, K)],
                                DI[b], SD[b])

    idx_d = {0: start_idx(0), 1: start_idx(1)}

    def fill(i, carry):
        ones[pl.ds(i * 16, 16)] = jnp.full((16,), 1.0, jnp.float32)
        return carry
    lax.fori_loop(0, K // 16, fill, 0)

    def zero(i, carry):
        dbuf[pl.ds(i * 16, 16)] = jnp.zeros((16,), jnp.float32)
        return carry
    lax.fori_loop(0, (SB + REM) // 16, zero, 0)
    pltpu.sync_copy(dbuf.at[pl.ds(0, SB)], acc.at[pl.ds(s * SB, SB)])

    @pl.when(s == NS - 1)
    def _remz():
        pltpu.sync_copy(dbuf.at[pl.ds(0, REM)], acc.at[pl.ds(ROFF, REM)])

    plsc.subcore_barrier()

    for k in range(CH):
        b = k & 1
        idx_d[k].wait()
        pltpu.sync_copy(ones, acc.at[DI[b]], add=True)
        if k + 2 < CH:
            idx_d[k + 2] = start_idx(k + 2)

    plsc.subcore_barrier()

    nmine = jnp.where(s == NS - 1, SB + REM, SB)
    pltpu.sync_copy(acc.at[pl.ds(s * SB, SB)], dbuf.at[pl.ds(0, SB)])

    @pl.when(s == NS - 1)
    def _remc():
        pltpu.sync_copy(acc.at[pl.ds(ROFF, REM)], dbuf.at[pl.ds(SB, REM)])

    def repl(i, carry):
        row = plsc.load_gather(dbuf, [jnp.full((16,), i, jnp.int32)])
        stage[pl.ds((i // PK) * 128 + (i % PK) * HID, HID)] = row
        return carry
    lax.fori_loop(0, nmine, repl, 0)

    pltpu.sync_copy(stage.at[pl.ds(0, RPS * 128)],
                    out_hbm.at[pl.ds(c * (NP * 128) + s * (RPS * 128),
                                     RPS * 128)])

    @pl.when(s == NS - 1)
    def _remo():
        pltpu.sync_copy(stage.at[pl.ds(RPS * 128, RREM * 128)],
                        out_hbm.at[pl.ds(c * (NP * 128) + NS * (RPS * 128),
                                         RREM * 128)])


def _bdiag_small(w):
    w128 = jnp.concatenate([w] * PK, axis=1)
    colb = lax.broadcasted_iota(jnp.int32, (HID, 128), 1) // HID
    rows = [jnp.where(colb == p, w128, 0.0) for p in range(PK)]
    return jnp.concatenate(rows, axis=0)


def _bdiag_first(w):
    w128 = jnp.concatenate([w] * PK, axis=1)
    colb = lax.broadcasted_iota(jnp.int32, (F_IN, 128), 1) // HID
    rows = [jnp.where(colb == p, w128, 0.0) for p in range(PK)]
    return jnp.concatenate(rows, axis=0)


def _tc_mm(x3, W1):
    def body(x_ref, w_ref, u_ref):
        wbig = _bdiag_first(w_ref[...])
        u_ref[...] = jnp.dot(x_ref[...], wbig,
                             preferred_element_type=jnp.float32)
    return pl.pallas_call(
        body,
        out_shape=jax.ShapeDtypeStruct((NP, 128), jnp.float32),
    )(x3, W1)


def _tc_dinv(u, degp):
    def body(u_ref, dp_ref, dinv_ref, g_ref):
        deg = dp_ref[0] + dp_ref[1] + 1.0
        dinv = lax.rsqrt(deg)
        dinv_ref[...] = dinv
        g_ref[...] = dinv * u_ref[...]
    return pl.pallas_call(
        body,
        out_shape=[
            jax.ShapeDtypeStruct((NP, 128), jnp.float32),
            jax.ShapeDtypeStruct((NP, 128), jnp.float32),
        ],
    )(u, degp)


def _tc_layer(aggp, g, dinv, b, W):
    def body(ap_ref, g_ref, dinv_ref, b_ref, w_ref, out_ref):
        dinv = dinv_ref[...]
        b128 = jnp.concatenate([b_ref[...]] * PK, axis=1)
        h = dinv * (ap_ref[0] + ap_ref[1] + g_ref[...]) + b128
        h = jnp.maximum(h, 0.0)
        wbd = _bdiag_small(w_ref[...])
        out_ref[...] = dinv * jnp.dot(h, wbd,
                                      preferred_element_type=jnp.float32)
    return pl.pallas_call(
        body,
        out_shape=jax.ShapeDtypeStruct((NP, 128), jnp.float32),
    )(aggp, g, dinv, b, W)


def _tc_final(aggp, g, dinv, b, batch2, Wl, bl):
    def body(ap_ref, g_ref, dinv_ref, b_ref, bt_ref, wl_ref, bl_ref, out_ref):
        b128 = jnp.concatenate([b_ref[...]] * PK, axis=1)
        h3 = dinv_ref[...] * (ap_ref[0] + ap_ref[1] + g_ref[...]) + b128
        pooled = jnp.zeros((G, HID), jnp.float32)
        for p in range(PK):
            hp = h3[:, p * HID:(p + 1) * HID]
            bp = bt_ref[:, p:p + 1]
            seg = (bp == lax.broadcasted_iota(jnp.int32, (NP, G), 1)
                   ).astype(jnp.float32)
            pooled += lax.dot_general(seg, hp, (((0,), (0,)), ((), ())),
                                      preferred_element_type=jnp.float32)
        out_ref[...] = jnp.dot(pooled, wl_ref[...],
                               preferred_element_type=jnp.float32) + bl_ref[...]

    return pl.pallas_call(
        body,
        out_shape=jax.ShapeDtypeStruct((G, C), jnp.float32),
    )(aggp, g, dinv, b, batch2, Wl, bl)


def kernel(x, edge_index, batch, W1, b1, W2, b2, W3, b3, Wl, bl):
    degp_p = _sc_deg(edge_index).reshape(NC, NP, 128)
    x3 = x.reshape(NP, PK * F_IN)
    u1_p = _tc_mm(x3, W1)
    dinv_p, g1_p = _tc_dinv(u1_p, degp_p)

    a1 = _sc_agg(edge_index, g1_p.reshape(N, HID)).reshape(NC, NP, 128)
    g2_p = _tc_layer(a1, g1_p, dinv_p, b1.reshape(1, HID), W2)

    a2 = _sc_agg(edge_index, g2_p.reshape(N, HID)).reshape(NC, NP, 128)
    g3_p = _tc_layer(a2, g2_p, dinv_p, b2.reshape(1, HID), W3)

    a3 = _sc_agg(edge_index, g3_p.reshape(N, HID)).reshape(NC, NP, 128)
    out = _tc_final(a3, g3_p, dinv_p,
                    b3.reshape(1, HID), batch.reshape(NP, PK),
                    Wl, bl.reshape(1, C))
    return out


# speedup vs baseline: 1.0173x; 1.0017x over previous
import functools

import jax
import jax.numpy as jnp
from jax import lax
from jax.experimental import pallas as pl
from jax.experimental.pallas import tpu as pltpu
from jax.experimental.pallas import tpu_sc as plsc

N = 10000
E = 320000
F_IN = 128
HID = 16
C = 10
G = 64

NC = 2
NS = 16
NW = NC * NS
EW = E // NW
K = 2000
CH = EW // K
SB = 624
REM = N - NS * SB
ROFF = NS * SB

PK = 128 // HID
NP = N // PK
_sc_mesh = plsc.VectorSubcoreMesh(core_axis_name="c", subcore_axis_name="s")
_sc_params = pltpu.CompilerParams(use_tc_tiling_on_sc=False,
                                  needs_layout_passes=False)


def _zero_acc_slice(zbuf, acc, s):
    def body(i, carry):
        zbuf[i, :] = jnp.zeros((HID,), jnp.float32)
        return carry
    lax.fori_loop(0, SB, body, 0)
    pltpu.sync_copy(zbuf.at[pl.ds(0, SB)], acc.at[pl.ds(s * SB, SB)])

    @pl.when(s == NS - 1)
    def _rem():
        pltpu.sync_copy(zbuf.at[pl.ds(0, REM)], acc.at[pl.ds(ROFF, REM)])


RPS = SB // PK
RREM = REM // PK
def _repack_out(abuf, stage, out_hbm, c, s, nmine):
    def body(i, carry):
        row = abuf[i, :]
        stage[pl.ds((i // PK) * 128 + (i % PK) * HID, HID)] = row
        return carry
    lax.fori_loop(0, nmine, body, 0)

    pltpu.sync_copy(stage.at[pl.ds(0, RPS * 128)],
                    out_hbm.at[pl.ds(c * (NP * 128) + s * (RPS * 128),
                                     RPS * 128)])

    @pl.when(s == NS - 1)
    def _rem():
        pltpu.sync_copy(stage.at[pl.ds(RPS * 128, RREM * 128)],
                        out_hbm.at[pl.ds(c * (NP * 128) + NS * (RPS * 128),
                                         RREM * 128)])


@functools.partial(
    pl.kernel,
    out_type=jax.ShapeDtypeStruct((NC * NP * 128,), jnp.float32),
    mesh=_sc_mesh,
    compiler_params=_sc_params,
    scratch_types=[
        pltpu.VMEM((K,), jnp.int32),
        pltpu.VMEM((K,), jnp.int32),
        pltpu.VMEM((K,), jnp.int32),
        pltpu.VMEM((K,), jnp.int32),
        pltpu.VMEM((K, HID), jnp.float32),
        pltpu.VMEM((K, HID), jnp.float32),
        pltpu.VMEM((SB + REM, HID), jnp.float32),
        pltpu.VMEM(((RPS + RREM) * 128,), jnp.float32),
        pltpu.VMEM_SHARED((N, HID), jnp.float32),
        pltpu.SemaphoreType.DMA,
        pltpu.SemaphoreType.DMA,
        pltpu.SemaphoreType.DMA,
        pltpu.SemaphoreType.DMA,
        pltpu.SemaphoreType.DMA,
        pltpu.SemaphoreType.DMA,
    ],
)
def _sc_agg(ei_hbm, g_hbm, out_hbm, si0, si1, di0, di1, r0, r1, zbuf, stage,
            acc, ss0, ss1, sd0, sd1, sg0, sg1):
    c = lax.axis_index("c")
    s = lax.axis_index("s")
    wid = s * NC + c
    base = wid * EW
    SI, DI, R = [si0, si1], [di0, di1], [r0, r1]
    SS, SD, SG = [ss0, ss1], [sd0, sd1], [sg0, sg1]

    def start_idx(k):
        b = k & 1
        return (pltpu.async_copy(ei_hbm.at[0, pl.ds(base + k * K, K)],
                                 SI[b], SS[b]),
                pltpu.async_copy(ei_hbm.at[1, pl.ds(base + k * K, K)],
                                 DI[b], SD[b]))

    idx_d = {0: start_idx(0), 1: start_idx(1)}
    idx_d[0][0].wait()
    g_d = {0: pltpu.async_copy(g_hbm.at[SI[0]], R[0], SG[0])}
    _zero_acc_slice(zbuf, acc, s)
    plsc.subcore_barrier()

    for k in range(CH):
        b = k & 1
        if k + 1 < CH:
            idx_d[k + 1][0].wait()
            g_d[k + 1] = pltpu.async_copy(g_hbm.at[SI[1 - b]], R[1 - b],
                                          SG[1 - b])
        g_d[k].wait()
        idx_d[k][1].wait()
        pltpu.sync_copy(R[b], acc.at[DI[b]], add=True)
        if k + 2 < CH:
            idx_d[k + 2] = start_idx(k + 2)

    plsc.subcore_barrier()

    nmine = jnp.where(s == NS - 1, SB + REM, SB)
    pltpu.sync_copy(acc.at[pl.ds(s * SB, SB)], zbuf.at[pl.ds(0, SB)])

    @pl.when(s == NS - 1)
    def _remc():
        pltpu.sync_copy(acc.at[pl.ds(ROFF, REM)], zbuf.at[pl.ds(SB, REM)])

    _repack_out(zbuf, stage, out_hbm, c, s, nmine)


@functools.partial(
    pl.kernel,
    out_type=jax.ShapeDtypeStruct((NC * NP * 128,), jnp.float32),
    mesh=_sc_mesh,
    compiler_params=_sc_params,
    scratch_types=[
        pltpu.VMEM((K,), jnp.int32),
        pltpu.VMEM((K,), jnp.int32),
        pltpu.VMEM((K,), jnp.float32),
        pltpu.VMEM((SB + REM,), jnp.float32),
        pltpu.VMEM(((RPS + RREM) * 128,), jnp.float32),
        pltpu.VMEM_SHARED((N,), jnp.float32),
        pltpu.SemaphoreType.DMA,
        pltpu.SemaphoreType.DMA,
    ],
)
def _sc_deg(ei_hbm, out_hbm, di0, di1, ones, dbuf, stage, acc, sd0, sd1):
    c = lax.axis_index("c")
    s = lax.axis_index("s")
    wid = s * NC + c
    base = wid * EW
    DI, SD = [di0, di1], [sd0, sd1]

    def start_idx(k):
        b = k & 1
        return pltpu.async_copy(ei_hbm.at[1, pl.ds(base + k * K, K)],
                                DI[b], SD[b])

    idx_d = {0: start_idx(0), 1: start_idx(1)}

    def fill(i, carry):
        ones[pl.ds(i * 16, 16)] = jnp.full((16,), 1.0, jnp.float32)
        return carry
    lax.fori_loop(0, K // 16, fill, 0)

    def zero(i, carry):
        dbuf[pl.ds(i * 16, 16)] = jnp.zeros((16,), jnp.float32)
        return carry
    lax.fori_loop(0, (SB + REM) // 16, zero, 0)
    pltpu.sync_copy(dbuf.at[pl.ds(0, SB)], acc.at[pl.ds(s * SB, SB)])

    @pl.when(s == NS - 1)
    def _remz():
        pltpu.sync_copy(dbuf.at[pl.ds(0, REM)], acc.at[pl.ds(ROFF, REM)])

    plsc.subcore_barrier()

    for k in range(CH):
        b = k & 1
        idx_d[k].wait()
        pltpu.sync_copy(ones, acc.at[DI[b]], add=True)
        if k + 2 < CH:
            idx_d[k + 2] = start_idx(k + 2)

    plsc.subcore_barrier()

    nmine = jnp.where(s == NS - 1, SB + REM, SB)
    pltpu.sync_copy(acc.at[pl.ds(s * SB, SB)], dbuf.at[pl.ds(0, SB)])

    @pl.when(s == NS - 1)
    def _remc():
        pltpu.sync_copy(acc.at[pl.ds(ROFF, REM)], dbuf.at[pl.ds(SB, REM)])

    def repl(i, carry):
        row = plsc.load_gather(dbuf, [jnp.full((16,), i, jnp.int32)])
        stage[pl.ds((i // PK) * 128 + (i % PK) * HID, HID)] = row
        return carry
    lax.fori_loop(0, nmine, repl, 0)

    pltpu.sync_copy(stage.at[pl.ds(0, RPS * 128)],
                    out_hbm.at[pl.ds(c * (NP * 128) + s * (RPS * 128),
                                     RPS * 128)])

    @pl.when(s == NS - 1)
    def _remo():
        pltpu.sync_copy(stage.at[pl.ds(RPS * 128, RREM * 128)],
                        out_hbm.at[pl.ds(c * (NP * 128) + NS * (RPS * 128),
                                         RREM * 128)])


def _bdiag_small(w):
    w128 = jnp.concatenate([w] * PK, axis=1)
    colb = lax.broadcasted_iota(jnp.int32, (HID, 128), 1) // HID
    rows = [jnp.where(colb == p, w128, 0.0) for p in range(PK)]
    return jnp.concatenate(rows, axis=0)


def _bdiag_first(w):
    w128 = jnp.concatenate([w] * PK, axis=1)
    colb = lax.broadcasted_iota(jnp.int32, (F_IN, 128), 1) // HID
    rows = [jnp.where(colb == p, w128, 0.0) for p in range(PK)]
    return jnp.concatenate(rows, axis=0)


def _tc_mm(x3, W1):
    def body(x_ref, w_ref, u_ref):
        wbig = _bdiag_first(w_ref[...])
        u_ref[...] = jnp.dot(x_ref[...], wbig,
                             preferred_element_type=jnp.float32)
    return pl.pallas_call(
        body,
        out_shape=jax.ShapeDtypeStruct((NP, 128), jnp.float32),
    )(x3, W1)


def _tc_dinv(u, degp):
    def body(u_ref, dp_ref, dinv_ref, g_ref):
        deg = dp_ref[0] + dp_ref[1] + 1.0
        dinv = lax.rsqrt(deg)
        dinv_ref[...] = dinv
        g_ref[...] = dinv * u_ref[...]
    return pl.pallas_call(
        body,
        out_shape=[
            jax.ShapeDtypeStruct((NP, 128), jnp.float32),
            jax.ShapeDtypeStruct((NP, 128), jnp.float32),
        ],
    )(u, degp)


def _tc_layer(aggp, g, dinv, b, W):
    def body(ap_ref, g_ref, dinv_ref, b_ref, w_ref, out_ref):
        dinv = dinv_ref[...]
        b128 = jnp.concatenate([b_ref[...]] * PK, axis=1)
        h = dinv * (ap_ref[0] + ap_ref[1] + g_ref[...]) + b128
        h = jnp.maximum(h, 0.0)
        wbd = _bdiag_small(w_ref[...])
        out_ref[...] = dinv * jnp.dot(h, wbd,
                                      preferred_element_type=jnp.float32)
    return pl.pallas_call(
        body,
        out_shape=jax.ShapeDtypeStruct((NP, 128), jnp.float32),
    )(aggp, g, dinv, b, W)


def _tc_final(aggp, g, dinv, b, batch2, Wl, bl):
    def body(ap_ref, g_ref, dinv_ref, b_ref, bt_ref, wl_ref, bl_ref, out_ref):
        b128 = jnp.concatenate([b_ref[...]] * PK, axis=1)
        h3 = dinv_ref[...] * (ap_ref[0] + ap_ref[1] + g_ref[...]) + b128
        pooled = jnp.zeros((G, HID), jnp.float32)
        for p in range(PK):
            hp = h3[:, p * HID:(p + 1) * HID]
            bp = bt_ref[:, p:p + 1]
            seg = (bp == lax.broadcasted_iota(jnp.int32, (NP, G), 1)
                   ).astype(jnp.float32)
            pooled += lax.dot_general(seg, hp, (((0,), (0,)), ((), ())),
                                      preferred_element_type=jnp.float32)
        out_ref[...] = jnp.dot(pooled, wl_ref[...],
                               preferred_element_type=jnp.float32) + bl_ref[...]

    return pl.pallas_call(
        body,
        out_shape=jax.ShapeDtypeStruct((G, C), jnp.float32),
    )(aggp, g, dinv, b, batch2, Wl, bl)


def kernel(x, edge_index, batch, W1, b1, W2, b2, W3, b3, Wl, bl):
    degp_p = _sc_deg(edge_index).reshape(NC, NP, 128)
    x3 = x.reshape(NP, PK * F_IN)
    u1_p = _tc_mm(x3, W1)
    dinv_p, g1_p = _tc_dinv(u1_p, degp_p)

    a1 = _sc_agg(edge_index, g1_p.reshape(N, HID)).reshape(NC, NP, 128)
    g2_p = _tc_layer(a1, g1_p, dinv_p, b1.reshape(1, HID), W2)

    a2 = _sc_agg(edge_index, g2_p.reshape(N, HID)).reshape(NC, NP, 128)
    g3_p = _tc_layer(a2, g2_p, dinv_p, b2.reshape(1, HID), W3)

    a3 = _sc_agg(edge_index, g3_p.reshape(N, HID)).reshape(NC, NP, 128)
    out = _tc_final(a3, g3_p, dinv_p,
                    b3.reshape(1, HID), batch.reshape(NP, PK),
                    Wl, bl.reshape(1, C))
    return out
